# baseline (speedup 1.0000x reference)
import functools
import jax
import jax.numpy as jnp
from jax import lax
from jax.experimental import pallas as pl
from jax.experimental.pallas import tpu as pltpu, tpu_sc as plsc

N_ROWS = 100000
K = 32
SLAB_ROWS = 400
G_PER_SLAB = SLAB_ROWS // 16
N_SLABS = N_ROWS // SLAB_ROWS
N_WORKERS = 32
MAX_SLABS_PER_W = (N_SLABS + N_WORKERS - 1) // N_WORKERS


def _body(neigh_hbm, consts_hbm, out_hbm, buf0, buf1, outbuf, cbuf, sem0, sem1):
    wid = lax.axis_index("s") * 2 + lax.axis_index("c")
    pltpu.sync_copy(consts_hbm, cbuf)
    c1 = cbuf[pl.ds(0, 16)]
    wv = cbuf[pl.ds(16, 16)]
    bg = cbuf[pl.ds(32, 16)]
    wr = cbuf[pl.ds(48, 16)]
    br = cbuf[pl.ds(64, 16)]
    lane = lax.iota(jnp.int32, 16)
    idx_base = lane * K

    def compute_slab(bslice, s):
        def group(g, carry):
            idx0 = idx_base + g * (16 * K)
            den = jnp.zeros((16,), jnp.float32)
            sv = jnp.zeros((16,), jnp.float32)
            for k in range(K):
                v = plsc.load_gather(bslice, [idx0 + k])
                t = v * c1
                e = jnp.exp(jnp.maximum(t, t * 0.2))
                den = den + e
                sv = sv + v * e
            o = (sv * wv) / (den + 1e-16) + bg
            r = jnp.where(o > 0, o, jnp.exp(o) - 1.0)
            outbuf[pl.ds(g * 16, 16)] = r * wr + br
            return carry

        lax.fori_loop(0, G_PER_SLAB, group, 0)
        pltpu.sync_copy(outbuf, out_hbm.at[pl.ds(s * SLAB_ROWS, SLAB_ROWS)])

    sems = (sem0, sem1)
    bufs = (buf0, buf1)

    def start(i):
        s = wid + N_WORKERS * i
        pltpu.async_copy(
            neigh_hbm.at[pl.ds(s * SLAB_ROWS * K, SLAB_ROWS * K)],
            bufs[i % 2], sems[i % 2])

    start(0)
    for i in range(MAX_SLABS_PER_W):
        s = wid + N_WORKERS * i
        if i + 1 < MAX_SLABS_PER_W:
            @pl.when(wid + N_WORKERS * (i + 1) < N_SLABS)
            def _():
                start(i + 1)

        @pl.when(s < N_SLABS)
        def _():
            b = i % 2
            pltpu.make_async_copy(
                neigh_hbm.at[pl.ds(s * SLAB_ROWS * K, SLAB_ROWS * K)],
                bufs[b], sems[b]).wait()
            compute_slab(bufs[b], s)


def kernel(query_emb, entity_emb, neighbors, W, a_src, a_tgt, b_gat, W_rank, b_rank):
    n = neighbors.shape[0]
    neigh_flat = neighbors.reshape(n * K)
    w0 = W[0, 0]
    consts = jnp.concatenate([
        jnp.full((16,), w0 * a_src[0, 0, 0], jnp.float32),
        jnp.full((16,), w0, jnp.float32),
        jnp.full((16,), b_gat[0], jnp.float32),
        jnp.full((16,), W_rank[0, 0], jnp.float32),
        jnp.full((16,), b_rank[0], jnp.float32),
    ])

    mesh = plsc.VectorSubcoreMesh(core_axis_name="c", subcore_axis_name="s")
    run = functools.partial(
        pl.kernel,
        mesh=mesh,
        compiler_params=pltpu.CompilerParams(needs_layout_passes=False),
        out_type=jax.ShapeDtypeStruct((n,), jnp.float32),
        scratch_types=[
            pltpu.VMEM((SLAB_ROWS * K,), jnp.float32),
            pltpu.VMEM((SLAB_ROWS * K,), jnp.float32),
            pltpu.VMEM((SLAB_ROWS,), jnp.float32),
            pltpu.VMEM((80,), jnp.float32),
            pltpu.SemaphoreType.DMA,
            pltpu.SemaphoreType.DMA,
        ],
    )(_body)
    out = run(neigh_flat, consts)
    return out.reshape(n, 1)


# speedup vs baseline: 1.3948x; 1.3948x over previous
import functools
import jax
import jax.numpy as jnp
from jax import lax
from jax.experimental import pallas as pl
from jax.experimental.pallas import tpu as pltpu, tpu_sc as plsc

N_ROWS = 100000
K = 32
SLAB_ROWS = 400
G_PER_SLAB = SLAB_ROWS // 16
N_SLABS = N_ROWS // SLAB_ROWS
N_WORKERS = 32
MAX_SLABS_PER_W = (N_SLABS + N_WORKERS - 1) // N_WORKERS


def _body(neigh_hbm, consts_hbm, out_hbm, buf0, buf1, outbuf, cbuf, sem0, sem1):
    wid = lax.axis_index("s") * 2 + lax.axis_index("c")
    pltpu.sync_copy(consts_hbm, cbuf)
    c1 = cbuf[pl.ds(0, 16)]
    wv = cbuf[pl.ds(16, 16)]
    bg = cbuf[pl.ds(32, 16)]
    wr = cbuf[pl.ds(48, 16)]
    br = cbuf[pl.ds(64, 16)]
    lane = lax.iota(jnp.int32, 16)
    idx_base = lane * K

    def compute_slab(bslice, s):
        def group(g, carry):
            idx0 = idx_base + g * (16 * K)
            den = [jnp.zeros((16,), jnp.float32) for _ in range(4)]
            sv = [jnp.zeros((16,), jnp.float32) for _ in range(4)]
            for k in range(K):
                v = plsc.load_gather(bslice, [idx0 + ((lane + k) & (K - 1))])
                t = v * c1
                e = jnp.exp(jnp.maximum(t, t * 0.2))
                den[k % 4] = den[k % 4] + e
                sv[k % 4] = sv[k % 4] + v * e
            dent = (den[0] + den[1]) + (den[2] + den[3])
            svt = (sv[0] + sv[1]) + (sv[2] + sv[3])
            o = (svt * wv) / (dent + 1e-16) + bg
            r = jnp.where(o > 0, o, jnp.exp(o) - 1.0)
            outbuf[pl.ds(g * 16, 16)] = r * wr + br
            return carry

        lax.fori_loop(0, G_PER_SLAB, group, 0)
        pltpu.sync_copy(outbuf, out_hbm.at[pl.ds(s * SLAB_ROWS, SLAB_ROWS)])

    sems = (sem0, sem1)
    bufs = (buf0, buf1)

    def start(i):
        s = wid + N_WORKERS * i
        pltpu.async_copy(
            neigh_hbm.at[pl.ds(s * SLAB_ROWS * K, SLAB_ROWS * K)],
            bufs[i % 2], sems[i % 2])

    start(0)
    for i in range(MAX_SLABS_PER_W):
        s = wid + N_WORKERS * i
        if i + 1 < MAX_SLABS_PER_W:
            @pl.when(wid + N_WORKERS * (i + 1) < N_SLABS)
            def _():
                start(i + 1)

        @pl.when(s < N_SLABS)
        def _():
            b = i % 2
            pltpu.make_async_copy(
                neigh_hbm.at[pl.ds(s * SLAB_ROWS * K, SLAB_ROWS * K)],
                bufs[b], sems[b]).wait()
            compute_slab(bufs[b], s)


def kernel(query_emb, entity_emb, neighbors, W, a_src, a_tgt, b_gat, W_rank, b_rank):
    n = neighbors.shape[0]
    neigh_flat = neighbors.reshape(n * K)
    w0 = W[0, 0]
    consts = jnp.concatenate([
        jnp.full((16,), w0 * a_src[0, 0, 0], jnp.float32),
        jnp.full((16,), w0, jnp.float32),
        jnp.full((16,), b_gat[0], jnp.float32),
        jnp.full((16,), W_rank[0, 0], jnp.float32),
        jnp.full((16,), b_rank[0], jnp.float32),
    ])

    mesh = plsc.VectorSubcoreMesh(core_axis_name="c", subcore_axis_name="s")
    run = functools.partial(
        pl.kernel,
        mesh=mesh,
        compiler_params=pltpu.CompilerParams(needs_layout_passes=False),
        out_type=jax.ShapeDtypeStruct((n,), jnp.float32),
        scratch_types=[
            pltpu.VMEM((SLAB_ROWS * K,), jnp.float32),
            pltpu.VMEM((SLAB_ROWS * K,), jnp.float32),
            pltpu.VMEM((SLAB_ROWS,), jnp.float32),
            pltpu.VMEM((80,), jnp.float32),
            pltpu.SemaphoreType.DMA,
            pltpu.SemaphoreType.DMA,
        ],
    )(_body)
    out = run(neigh_flat, consts)
    return out.reshape(n, 1)


# speedup vs baseline: 1.4989x; 1.0746x over previous
import functools
import jax
import jax.numpy as jnp
from jax import lax
from jax.experimental import pallas as pl
from jax.experimental.pallas import tpu as pltpu, tpu_sc as plsc

N_ROWS = 100000
K = 32
SLAB_ROWS = 800
G_PER_SLAB = SLAB_ROWS // 16
N_SLABS = N_ROWS // SLAB_ROWS
N_WORKERS = 32
MAX_SLABS_PER_W = (N_SLABS + N_WORKERS - 1) // N_WORKERS
GSTRIDE = 16 * K


def _body(neigh_hbm, consts_hbm, out_hbm, buf0, buf1, outbuf, cbuf, sem0, sem1):
    wid = lax.axis_index("s") * 2 + lax.axis_index("c")
    pltpu.sync_copy(consts_hbm, cbuf)
    c1 = cbuf[pl.ds(0, 16)]
    wv = cbuf[pl.ds(16, 16)]
    bg = cbuf[pl.ds(32, 16)]
    wr = cbuf[pl.ds(48, 16)]
    br = cbuf[pl.ds(64, 16)]
    lane = lax.iota(jnp.int32, 16)
    idx_base = lane * K

    def compute_slab(bslice, s):
        def pairgroup(j, carry):
            idx0a = idx_base + j * (2 * GSTRIDE)
            idx0b = idx0a + GSTRIDE
            dena = [jnp.zeros((16,), jnp.float32) for _ in range(4)]
            svna = [jnp.zeros((16,), jnp.float32) for _ in range(4)]
            denb = [jnp.zeros((16,), jnp.float32) for _ in range(4)]
            svnb = [jnp.zeros((16,), jnp.float32) for _ in range(4)]
            for k in range(K):
                sk = (lane + k) & (K - 1)
                va = plsc.load_gather(bslice, [idx0a + sk])
                vb = plsc.load_gather(bslice, [idx0b + sk])
                ta = va * c1
                tb = vb * c1
                ea = jnp.exp(jnp.maximum(ta, ta * 0.2))
                eb = jnp.exp(jnp.maximum(tb, tb * 0.2))
                dena[k % 4] = dena[k % 4] + ea
                svna[k % 4] = svna[k % 4] + va * ea
                denb[k % 4] = denb[k % 4] + eb
                svnb[k % 4] = svnb[k % 4] + vb * eb
            da = (dena[0] + dena[1]) + (dena[2] + dena[3])
            sa = (svna[0] + svna[1]) + (svna[2] + svna[3])
            db = (denb[0] + denb[1]) + (denb[2] + denb[3])
            sb = (svnb[0] + svnb[1]) + (svnb[2] + svnb[3])
            oa = (sa * wv) / (da + 1e-16) + bg
            ob = (sb * wv) / (db + 1e-16) + bg
            ra = jnp.where(oa > 0, oa, jnp.exp(oa) - 1.0)
            rb = jnp.where(ob > 0, ob, jnp.exp(ob) - 1.0)
            outbuf[pl.ds(j * 32, 16)] = ra * wr + br
            outbuf[pl.ds(j * 32 + 16, 16)] = rb * wr + br
            return carry

        lax.fori_loop(0, G_PER_SLAB // 2, pairgroup, 0)
        pltpu.sync_copy(outbuf, out_hbm.at[pl.ds(s * SLAB_ROWS, SLAB_ROWS)])

    sems = (sem0, sem1)
    bufs = (buf0, buf1)

    def start(i):
        s = wid + N_WORKERS * i
        pltpu.async_copy(
            neigh_hbm.at[pl.ds(s * SLAB_ROWS * K, SLAB_ROWS * K)],
            bufs[i % 2], sems[i % 2])

    start(0)
    for i in range(MAX_SLABS_PER_W):
        s = wid + N_WORKERS * i
        if i + 1 < MAX_SLABS_PER_W:
            @pl.when(wid + N_WORKERS * (i + 1) < N_SLABS)
            def _():
                start(i + 1)

        @pl.when(s < N_SLABS)
        def _():
            b = i % 2
            pltpu.make_async_copy(
                neigh_hbm.at[pl.ds(s * SLAB_ROWS * K, SLAB_ROWS * K)],
                bufs[b], sems[b]).wait()
            compute_slab(bufs[b], s)


def kernel(query_emb, entity_emb, neighbors, W, a_src, a_tgt, b_gat, W_rank, b_rank):
    n = neighbors.shape[0]
    neigh_flat = neighbors.reshape(n * K)
    w0 = W[0, 0]
    consts = jnp.concatenate([
        jnp.full((16,), w0 * a_src[0, 0, 0], jnp.float32),
        jnp.full((16,), w0, jnp.float32),
        jnp.full((16,), b_gat[0], jnp.float32),
        jnp.full((16,), W_rank[0, 0], jnp.float32),
        jnp.full((16,), b_rank[0], jnp.float32),
    ])

    mesh = plsc.VectorSubcoreMesh(core_axis_name="c", subcore_axis_name="s")
    run = functools.partial(
        pl.kernel,
        mesh=mesh,
        compiler_params=pltpu.CompilerParams(needs_layout_passes=False),
        out_type=jax.ShapeDtypeStruct((n,), jnp.float32),
        scratch_types=[
            pltpu.VMEM((SLAB_ROWS * K,), jnp.float32),
            pltpu.VMEM((SLAB_ROWS * K,), jnp.float32),
            pltpu.VMEM((SLAB_ROWS,), jnp.float32),
            pltpu.VMEM((80,), jnp.float32),
            pltpu.SemaphoreType.DMA,
            pltpu.SemaphoreType.DMA,
        ],
    )(_body)
    out = run(neigh_flat, consts)
    return out.reshape(n, 1)


# speedup vs baseline: 1.5023x; 1.0023x over previous
import functools
import jax
import jax.numpy as jnp
from jax import lax
from jax.experimental import pallas as pl
from jax.experimental.pallas import tpu as pltpu, tpu_sc as plsc

N_ROWS = 100000
K = 32
SLAB_ROWS = 800
G_PER_SLAB = SLAB_ROWS // 16
N_SLABS = N_ROWS // SLAB_ROWS
N_WORKERS = 32
MAX_SLABS_PER_W = (N_SLABS + N_WORKERS - 1) // N_WORKERS
GSTRIDE = 16 * K


def _body(neigh_hbm, consts_hbm, out_hbm, buf0, buf1, outbuf0, outbuf1, cbuf, sem0, sem1, osem0, osem1):
    wid = lax.axis_index("s") * 2 + lax.axis_index("c")
    pltpu.sync_copy(consts_hbm, cbuf)
    c1 = cbuf[pl.ds(0, 16)]
    wv = cbuf[pl.ds(16, 16)]
    bg = cbuf[pl.ds(32, 16)]
    wr = cbuf[pl.ds(48, 16)]
    br = cbuf[pl.ds(64, 16)]
    lane = lax.iota(jnp.int32, 16)
    idx_base = lane * K

    def compute_slab(bslice, outbuf, s):
        def pairgroup(j, carry):
            idx0a = idx_base + j * (2 * GSTRIDE)
            idx0b = idx0a + GSTRIDE
            dena = [jnp.zeros((16,), jnp.float32) for _ in range(4)]
            svna = [jnp.zeros((16,), jnp.float32) for _ in range(4)]
            denb = [jnp.zeros((16,), jnp.float32) for _ in range(4)]
            svnb = [jnp.zeros((16,), jnp.float32) for _ in range(4)]
            for k in range(K):
                sk = (lane + k) & (K - 1)
                va = plsc.load_gather(bslice, [idx0a + sk])
                vb = plsc.load_gather(bslice, [idx0b + sk])
                ta = va * c1
                tb = vb * c1
                ea = jnp.exp(jnp.maximum(ta, ta * 0.2))
                eb = jnp.exp(jnp.maximum(tb, tb * 0.2))
                dena[k % 4] = dena[k % 4] + ea
                svna[k % 4] = svna[k % 4] + va * ea
                denb[k % 4] = denb[k % 4] + eb
                svnb[k % 4] = svnb[k % 4] + vb * eb
            da = (dena[0] + dena[1]) + (dena[2] + dena[3])
            sa = (svna[0] + svna[1]) + (svna[2] + svna[3])
            db = (denb[0] + denb[1]) + (denb[2] + denb[3])
            sb = (svnb[0] + svnb[1]) + (svnb[2] + svnb[3])
            oa = (sa * wv) / (da + 1e-16) + bg
            ob = (sb * wv) / (db + 1e-16) + bg
            ra = jnp.where(oa > 0, oa, jnp.exp(oa) - 1.0)
            rb = jnp.where(ob > 0, ob, jnp.exp(ob) - 1.0)
            outbuf[pl.ds(j * 32, 16)] = ra * wr + br
            outbuf[pl.ds(j * 32 + 16, 16)] = rb * wr + br
            return carry

        lax.fori_loop(0, G_PER_SLAB // 2, pairgroup, 0)

    sems = (sem0, sem1)
    bufs = (buf0, buf1)
    osems = (osem0, osem1)
    outbufs = (outbuf0, outbuf1)

    def start(i):
        s = wid + N_WORKERS * i
        pltpu.async_copy(
            neigh_hbm.at[pl.ds(s * SLAB_ROWS * K, SLAB_ROWS * K)],
            bufs[i % 2], sems[i % 2])

    start(0)
    for i in range(MAX_SLABS_PER_W):
        s = wid + N_WORKERS * i
        b = i % 2
        if i + 1 < MAX_SLABS_PER_W:
            @pl.when(wid + N_WORKERS * (i + 1) < N_SLABS)
            def _():
                start(i + 1)

        @pl.when(s < N_SLABS)
        def _():
            pltpu.make_async_copy(
                neigh_hbm.at[pl.ds(s * SLAB_ROWS * K, SLAB_ROWS * K)],
                bufs[b], sems[b]).wait()
            if i >= 2:
                pltpu.make_async_copy(
                    outbufs[b],
                    out_hbm.at[pl.ds((s - 2 * N_WORKERS) * SLAB_ROWS,
                                     SLAB_ROWS)],
                    osems[b]).wait()
            compute_slab(bufs[b], outbufs[b], s)
            pltpu.async_copy(
                outbufs[b], out_hbm.at[pl.ds(s * SLAB_ROWS, SLAB_ROWS)],
                osems[b])

    for i in range(MAX_SLABS_PER_W):
        s = wid + N_WORKERS * i
        b = i % 2

        @pl.when(jnp.logical_and(s < N_SLABS, s + 2 * N_WORKERS >= N_SLABS))
        def _():
            pltpu.make_async_copy(
                outbufs[b], out_hbm.at[pl.ds(s * SLAB_ROWS, SLAB_ROWS)],
                osems[b]).wait()


def kernel(query_emb, entity_emb, neighbors, W, a_src, a_tgt, b_gat, W_rank, b_rank):
    n = neighbors.shape[0]
    neigh_flat = neighbors.reshape(n * K)
    w0 = W[0, 0]
    consts = jnp.concatenate([
        jnp.full((16,), w0 * a_src[0, 0, 0], jnp.float32),
        jnp.full((16,), w0, jnp.float32),
        jnp.full((16,), b_gat[0], jnp.float32),
        jnp.full((16,), W_rank[0, 0], jnp.float32),
        jnp.full((16,), b_rank[0], jnp.float32),
    ])

    mesh = plsc.VectorSubcoreMesh(core_axis_name="c", subcore_axis_name="s")
    run = functools.partial(
        pl.kernel,
        mesh=mesh,
        compiler_params=pltpu.CompilerParams(needs_layout_passes=False),
        out_type=jax.ShapeDtypeStruct((n,), jnp.float32),
        scratch_types=[
            pltpu.VMEM((SLAB_ROWS * K,), jnp.float32),
            pltpu.VMEM((SLAB_ROWS * K,), jnp.float32),
            pltpu.VMEM((SLAB_ROWS,), jnp.float32),
            pltpu.VMEM((SLAB_ROWS,), jnp.float32),
            pltpu.VMEM((80,), jnp.float32),
            pltpu.SemaphoreType.DMA,
            pltpu.SemaphoreType.DMA,
            pltpu.SemaphoreType.DMA,
            pltpu.SemaphoreType.DMA,
        ],
    )(_body)
    out = run(neigh_flat, consts)
    return out.reshape(n, 1)


# speedup vs baseline: 1.5256x; 1.0156x over previous
import functools
import jax
import jax.numpy as jnp
from jax import lax
from jax.experimental import pallas as pl
from jax.experimental.pallas import tpu as pltpu, tpu_sc as plsc

N_ROWS = 100000
K = 32
SLAB_ROWS = 800
G_PER_SLAB = SLAB_ROWS // 16
N_SLABS = N_ROWS // SLAB_ROWS
N_WORKERS = 32
MAX_SLABS_PER_W = (N_SLABS + N_WORKERS - 1) // N_WORKERS
FAT = 4
FAT_COLS = FAT * K
FAT_PER_SLAB = SLAB_ROWS // FAT
CHUNKS = ((0, 104), (104, 96))
IDX_PAD = 208


def _body(neigh_hbm, consts_hbm, out_hbm, buf0, buf1, outbuf0, outbuf1,
          idx0, idx1, cbuf, sem0, sem1, osem0, osem1):
    wid = lax.axis_index("s") * 2 + lax.axis_index("c")
    pltpu.sync_copy(consts_hbm, cbuf)
    c1 = cbuf[pl.ds(0, 16)]
    wv = cbuf[pl.ds(16, 16)]
    bg = cbuf[pl.ds(32, 16)]
    wr = cbuf[pl.ds(48, 16)]
    br = cbuf[pl.ds(64, 16)]
    lane = lax.iota(jnp.int32, 16)

    sems = (sem0, sem1)
    bufs = (buf0, buf1)
    osems = (osem0, osem1)
    outbufs = (outbuf0, outbuf1)
    idxs = (idx0, idx1)

    lane4 = lax.shift_right_logical(lane, 2)
    lane3_32 = (lane & 3) * K

    def compute_slab(buf2d, outbuf, s):
        def pairgroup(j, carry):
            fata = lane4 + j * 8
            fatb = fata + 4
            dena = [jnp.zeros((16,), jnp.float32) for _ in range(4)]
            svna = [jnp.zeros((16,), jnp.float32) for _ in range(4)]
            denb = [jnp.zeros((16,), jnp.float32) for _ in range(4)]
            svnb = [jnp.zeros((16,), jnp.float32) for _ in range(4)]
            for k in range(K):
                col = lane3_32 + ((lane + k) & (K - 1))
                va = plsc.load_gather(buf2d, [fata, col])
                vb = plsc.load_gather(buf2d, [fatb, col])
                ta = va * c1
                tb = vb * c1
                ea = jnp.exp(jnp.maximum(ta, ta * 0.2))
                eb = jnp.exp(jnp.maximum(tb, tb * 0.2))
                dena[k % 4] = dena[k % 4] + ea
                svna[k % 4] = svna[k % 4] + va * ea
                denb[k % 4] = denb[k % 4] + eb
                svnb[k % 4] = svnb[k % 4] + vb * eb
            da = (dena[0] + dena[1]) + (dena[2] + dena[3])
            sa = (svna[0] + svna[1]) + (svna[2] + svna[3])
            db = (denb[0] + denb[1]) + (denb[2] + denb[3])
            sb = (svnb[0] + svnb[1]) + (svnb[2] + svnb[3])
            oa = (sa * wv) / (da + 1e-16) + bg
            ob = (sb * wv) / (db + 1e-16) + bg
            ra = jnp.where(oa > 0, oa, jnp.exp(oa) - 1.0)
            rb = jnp.where(ob > 0, ob, jnp.exp(ob) - 1.0)
            outbuf[pl.ds(j * 32, 16)] = ra * wr + br
            outbuf[pl.ds(j * 32 + 16, 16)] = rb * wr + br
            return carry

        lax.fori_loop(0, G_PER_SLAB // 2, pairgroup, 0)

    def start(i):
        s = wid + N_WORKERS * i
        b = i % 2
        idxr = idxs[b]
        base = s * FAT_PER_SLAB
        for t in range(IDX_PAD // 16):
            idxr[pl.ds(t * 16, 16)] = base + t * 16 + lane
        for off, sz in CHUNKS:
            pltpu.async_copy(
                neigh_hbm.at[idxr.at[pl.ds(off, sz)]],
                bufs[b].at[pl.ds(off, sz), :],
                sems[b])

    def wait_in(i):
        b = i % 2
        idxr = idxs[b]
        for off, sz in CHUNKS:
            pltpu.make_async_copy(
                neigh_hbm.at[idxr.at[pl.ds(off, sz)]],
                bufs[b].at[pl.ds(off, sz), :],
                sems[b]).wait()

    start(0)
    for i in range(MAX_SLABS_PER_W):
        s = wid + N_WORKERS * i
        b = i % 2
        if i + 1 < MAX_SLABS_PER_W:
            @pl.when(wid + N_WORKERS * (i + 1) < N_SLABS)
            def _():
                start(i + 1)

        @pl.when(s < N_SLABS)
        def _():
            wait_in(i)
            if i >= 2:
                pltpu.make_async_copy(
                    outbufs[b],
                    out_hbm.at[pl.ds((s - 2 * N_WORKERS) * SLAB_ROWS,
                                     SLAB_ROWS)],
                    osems[b]).wait()
            compute_slab(bufs[b], outbufs[b], s)
            pltpu.async_copy(
                outbufs[b], out_hbm.at[pl.ds(s * SLAB_ROWS, SLAB_ROWS)],
                osems[b])

    for i in range(MAX_SLABS_PER_W):
        s = wid + N_WORKERS * i
        b = i % 2

        @pl.when(jnp.logical_and(s < N_SLABS, s + 2 * N_WORKERS >= N_SLABS))
        def _():
            pltpu.make_async_copy(
                outbufs[b], out_hbm.at[pl.ds(s * SLAB_ROWS, SLAB_ROWS)],
                osems[b]).wait()


def kernel(query_emb, entity_emb, neighbors, W, a_src, a_tgt, b_gat, W_rank, b_rank):
    n = neighbors.shape[0]
    neigh2d = neighbors.reshape(n * K // FAT_COLS, FAT_COLS)
    w0 = W[0, 0]
    consts = jnp.concatenate([
        jnp.full((16,), w0 * a_src[0, 0, 0], jnp.float32),
        jnp.full((16,), w0, jnp.float32),
        jnp.full((16,), b_gat[0], jnp.float32),
        jnp.full((16,), W_rank[0, 0], jnp.float32),
        jnp.full((16,), b_rank[0], jnp.float32),
    ])

    mesh = plsc.VectorSubcoreMesh(core_axis_name="c", subcore_axis_name="s")
    run = functools.partial(
        pl.kernel,
        mesh=mesh,
        compiler_params=pltpu.CompilerParams(needs_layout_passes=False),
        out_type=jax.ShapeDtypeStruct((n,), jnp.float32),
        scratch_types=[
            pltpu.VMEM((FAT_PER_SLAB, FAT_COLS), jnp.float32),
            pltpu.VMEM((FAT_PER_SLAB, FAT_COLS), jnp.float32),
            pltpu.VMEM((SLAB_ROWS,), jnp.float32),
            pltpu.VMEM((SLAB_ROWS,), jnp.float32),
            pltpu.VMEM((IDX_PAD,), jnp.int32),
            pltpu.VMEM((IDX_PAD,), jnp.int32),
            pltpu.VMEM((80,), jnp.float32),
            pltpu.SemaphoreType.DMA,
            pltpu.SemaphoreType.DMA,
            pltpu.SemaphoreType.DMA,
            pltpu.SemaphoreType.DMA,
        ],
    )(_body)
    out = run(neigh2d, consts)
    return out.reshape(n, 1)
